# Initial kernel scaffold; baseline (speedup 1.0000x reference)
import jax
import jax.numpy as jnp
from jax.experimental import pallas as pl


def kernel(high_dim_features, low_dim_features, edge_index, g1, b1, Wm, bm, g2, b2, Wg, bg, Wc, bc):
    raise NotImplementedError("write your pallas kernel here")


# speedup vs baseline: 13.0321x; 13.0321x over previous
import functools

import jax
import jax.numpy as jnp
from jax import lax
from jax.experimental import pallas as pl
from jax.experimental.pallas import tpu as pltpu
from jax.experimental.pallas import tpu_sc as plsc

NC = 2
NS = 16
NW = NC * NS
CHUNK = 128
NP = 10240


def _deg_body(cpt, col_hbm, out_hbm, col_v, ones_v, deg_sh, sem):
    del sem
    c = lax.axis_index("c")
    s = lax.axis_index("s")
    w = s * NC + c
    rpt = NP // NS

    for j in range(8):
        ones_v[pl.ds(16 * j, 16)] = jnp.zeros((16,), jnp.float32)
    for j in range(rpt // CHUNK):
        pltpu.sync_copy(ones_v, deg_sh.at[pl.ds(s * rpt + j * CHUNK, CHUNK)])

    pltpu.sync_copy(col_hbm.at[pl.ds(w * cpt, cpt)], col_v)
    for j in range(8):
        ones_v[pl.ds(16 * j, 16)] = jnp.ones((16,), jnp.float32)
    plsc.subcore_barrier()

    def body(j, carry):
        pltpu.sync_copy(ones_v, deg_sh.at[col_v.at[j]], add=True)
        return carry

    lax.fori_loop(0, cpt, body, 0)
    plsc.subcore_barrier()
    pltpu.sync_copy(deg_sh.at[pl.ds(s * rpt, rpt)],
                    out_hbm.at[c, pl.ds(s * rpt, rpt)])


def _deg_call(col2d, cpt):
    kfn = pl.kernel(
        functools.partial(_deg_body, cpt),
        out_type=jax.ShapeDtypeStruct((NC, NP), jnp.float32),
        mesh=plsc.VectorSubcoreMesh(core_axis_name="c", subcore_axis_name="s"),
        scratch_types=[
            pltpu.VMEM((cpt, CHUNK), jnp.int32),
            pltpu.VMEM((CHUNK,), jnp.float32),
            pltpu.VMEM_SHARED((NP,), jnp.float32),
            pltpu.SemaphoreType.DMA,
        ],
    )
    return kfn(col2d)


def _gs_body(cpt, xws_hbm, row_hbm, col_hbm, out_hbm,
             row_v, col_v, rows_v, zb, acc_sh, sem):
    c = lax.axis_index("c")
    s = lax.axis_index("s")
    w = s * NC + c
    rpt = NP // NS

    for i in range(8):
        for j in range(8):
            zb[i, pl.ds(16 * j, 16)] = jnp.zeros((16,), jnp.float32)
    for j in range(rpt // 8):
        pltpu.sync_copy(zb, acc_sh.at[pl.ds(s * rpt + j * 8, 8)])

    pltpu.sync_copy(row_hbm.at[pl.ds(w * cpt, cpt)], row_v)
    pltpu.sync_copy(col_hbm.at[pl.ds(w * cpt, cpt)], col_v)
    plsc.subcore_barrier()

    def body(j, carry):
        pltpu.async_copy(xws_hbm.at[row_v.at[j]], rows_v, sem).wait()
        pltpu.sync_copy(rows_v, acc_sh.at[col_v.at[j]], add=True)
        return carry

    lax.fori_loop(0, cpt, body, 0)
    plsc.subcore_barrier()
    pltpu.sync_copy(acc_sh.at[pl.ds(s * rpt, rpt)],
                    out_hbm.at[c, pl.ds(s * rpt, rpt)])


def _gs_call(xws, row2d, col2d, cpt):
    kfn = pl.kernel(
        functools.partial(_gs_body, cpt),
        out_type=jax.ShapeDtypeStruct((NC, NP, 128), jnp.float32),
        mesh=plsc.VectorSubcoreMesh(core_axis_name="c", subcore_axis_name="s"),
        scratch_types=[
            pltpu.VMEM((cpt, CHUNK), jnp.int32),
            pltpu.VMEM((cpt, CHUNK), jnp.int32),
            pltpu.VMEM((CHUNK, 128), jnp.float32),
            pltpu.VMEM((8, 128), jnp.float32),
            pltpu.VMEM_SHARED((NP, 128), jnp.float32),
            pltpu.SemaphoreType.DMA,
        ],
    )
    return kfn(xws, row2d, col2d)


def _dense1_body(x_ref, dT_ref, g1_ref, b1_ref, Wm_ref, bm_ref,
                 g2_ref, b2_ref, Wg_ref, xws_ref):
    x = x_ref[...]
    mu = jnp.mean(x, axis=0, keepdims=True)
    xc = x - mu
    var = jnp.mean(xc * xc, axis=0, keepdims=True)
    x = g1_ref[...] * xc * lax.rsqrt(var + 1e-5) + b1_ref[...]
    x = jnp.maximum(
        jnp.dot(x, Wm_ref[...], preferred_element_type=jnp.float32)
        + bm_ref[...], 0.0)
    mu2 = jnp.mean(x, axis=0, keepdims=True)
    xc2 = x - mu2
    var2 = jnp.mean(xc2 * xc2, axis=0, keepdims=True)
    x = g2_ref[...] * xc2 * lax.rsqrt(var2 + 1e-5) + b2_ref[...]
    xw = jnp.dot(x, Wg_ref[...], preferred_element_type=jnp.float32)
    deg = dT_ref[:, 0:1] + dT_ref[:, 1:2] + 1.0
    xws_ref[...] = lax.rsqrt(deg) * xw


def _combine_body(n, p_ref, xws_ref, dT_ref, bg_ref, Wc_ref, bc_ref, out_ref):
    tot = p_ref[0, 0:n, :] + p_ref[1, 0:n, :] + xws_ref[...]
    deg = dT_ref[:, 0:1] + dT_ref[:, 1:2] + 1.0
    h = jnp.maximum(lax.rsqrt(deg) * tot + bg_ref[...], 0.0)
    logits = jnp.dot(h, Wc_ref[...], preferred_element_type=jnp.float32) \
        + bc_ref[...]
    m = jnp.max(logits, axis=1, keepdims=True)
    lse = jnp.log(jnp.sum(jnp.exp(logits - m), axis=1, keepdims=True))
    out_ref[...] = logits - m - lse


def kernel(high_dim_features, low_dim_features, edge_index,
           g1, b1, Wm, bm, g2, b2, Wg, bg, Wc, bc):
    del low_dim_features
    n, d = high_dim_features.shape
    e = edge_index.shape[1]
    cpt = -(-e // (NW * CHUNK))
    cpt = -(-cpt // 8) * 8
    e_pad = NW * cpt * CHUNK

    row = jnp.concatenate(
        [edge_index[0], jnp.zeros((e_pad - e,), jnp.int32)]).reshape(-1, CHUNK)
    col = jnp.concatenate(
        [edge_index[1],
         jnp.full((e_pad - e,), NP - 1, jnp.int32)]).reshape(-1, CHUNK)

    degs = _deg_call(col, cpt)
    degsT = jnp.swapaxes(degs, 0, 1)[:n]

    xws = pl.pallas_call(
        _dense1_body,
        out_shape=jax.ShapeDtypeStruct((n, d), jnp.float32),
    )(high_dim_features, degsT,
      g1.reshape(1, -1), b1.reshape(1, -1), Wm, bm.reshape(1, -1),
      g2.reshape(1, -1), b2.reshape(1, -1), Wg)

    parts = _gs_call(xws, row, col, cpt)

    out = pl.pallas_call(
        functools.partial(_combine_body, n),
        out_shape=jax.ShapeDtypeStruct((n, Wc.shape[1]), jnp.float32),
    )(parts, xws, degsT, bg.reshape(1, -1), Wc, bc.reshape(1, -1))
    return out


# speedup vs baseline: 34.7707x; 2.6681x over previous
import functools

import jax
import jax.numpy as jnp
from jax import lax
from jax.experimental import pallas as pl
from jax.experimental.pallas import tpu as pltpu
from jax.experimental.pallas import tpu_sc as plsc

NC = 2
NS = 16
NW = NC * NS
CHUNK = 128
SB = 16
NP = 10240


def _deg_body(cpt, col_hbm, out_hbm, col_v, ones_v, deg_sh, sem):
    del sem
    c = lax.axis_index("c")
    s = lax.axis_index("s")
    w = s * NC + c
    rpt = NP // NS

    for j in range(8):
        ones_v[pl.ds(16 * j, 16)] = jnp.zeros((16,), jnp.float32)
    for j in range(rpt // CHUNK):
        pltpu.sync_copy(ones_v, deg_sh.at[pl.ds(s * rpt + j * CHUNK, CHUNK)])

    pltpu.sync_copy(col_hbm.at[pl.ds(w * cpt, cpt)], col_v)
    for j in range(8):
        ones_v[pl.ds(16 * j, 16)] = jnp.ones((16,), jnp.float32)
    plsc.subcore_barrier()

    def body(j, carry):
        pltpu.sync_copy(ones_v, deg_sh.at[col_v.at[j]], add=True)
        return carry

    lax.fori_loop(0, cpt, body, 0)
    plsc.subcore_barrier()
    pltpu.sync_copy(deg_sh.at[pl.ds(s * rpt, rpt)],
                    out_hbm.at[c, pl.ds(s * rpt, rpt)])


def _deg_call(col2d, cpt):
    kfn = pl.kernel(
        functools.partial(_deg_body, cpt),
        out_type=jax.ShapeDtypeStruct((NC, NP), jnp.float32),
        mesh=plsc.VectorSubcoreMesh(core_axis_name="c", subcore_axis_name="s"),
        scratch_types=[
            pltpu.VMEM((cpt, CHUNK), jnp.int32),
            pltpu.VMEM((CHUNK,), jnp.float32),
            pltpu.VMEM_SHARED((NP,), jnp.float32),
            pltpu.SemaphoreType.DMA,
        ],
    )
    return kfn(col2d)


def _gs_body(cpt, xws_hbm, row_hbm, col_hbm, out_hbm,
             row_v, col_v, rows_v0, rows_v1, zb, acc_sh, sem0, sem1):
    c = lax.axis_index("c")
    s = lax.axis_index("s")
    w = s * NC + c
    rpt = NP // NS
    bufs = (rows_v0, rows_v1)
    sems = (sem0, sem1)

    for i in range(8):
        for j in range(8):
            zb[i, pl.ds(16 * j, 16)] = jnp.zeros((16,), jnp.float32)
    for j in range(rpt // 8):
        pltpu.sync_copy(zb, acc_sh.at[pl.ds(s * rpt + j * 8, 8)])

    plsc.subcore_barrier()

    def stage(t, carry):
        base = w * cpt + t * SB
        pltpu.sync_copy(row_hbm.at[pl.ds(base, SB)], row_v)
        pltpu.sync_copy(col_hbm.at[pl.ds(base, SB)], col_v)
        pltpu.async_copy(xws_hbm.at[row_v.at[0]], bufs[0], sems[0])

        def body(i, carry2):
            for b in range(2):
                j = 2 * i + b

                @pl.when(j + 1 < SB)
                def _():
                    pltpu.async_copy(xws_hbm.at[row_v.at[j + 1]],
                                     bufs[1 - b], sems[1 - b])

                pltpu.make_async_copy(xws_hbm.at[row_v.at[j]],
                                      bufs[b], sems[b]).wait()
                pltpu.sync_copy(bufs[b], acc_sh.at[col_v.at[j]], add=True)
            return carry2

        lax.fori_loop(0, SB // 2, body, 0)
        return carry

    lax.fori_loop(0, cpt // SB, stage, 0)
    plsc.subcore_barrier()
    pltpu.sync_copy(acc_sh.at[pl.ds(s * rpt, rpt)],
                    out_hbm.at[c, pl.ds(s * rpt, rpt)])


def _gs_call(xws, row2d, col2d, cpt):
    kfn = pl.kernel(
        functools.partial(_gs_body, cpt),
        out_type=jax.ShapeDtypeStruct((NC, NP, 128), jnp.float32),
        mesh=plsc.VectorSubcoreMesh(core_axis_name="c", subcore_axis_name="s"),
        scratch_types=[
            pltpu.VMEM((SB, CHUNK), jnp.int32),
            pltpu.VMEM((SB, CHUNK), jnp.int32),
            pltpu.VMEM((CHUNK, 128), jnp.float32),
            pltpu.VMEM((CHUNK, 128), jnp.float32),
            pltpu.VMEM((8, 128), jnp.float32),
            pltpu.VMEM_SHARED((NP, 128), jnp.float32),
            pltpu.SemaphoreType.DMA,
            pltpu.SemaphoreType.DMA,
        ],
    )
    return kfn(xws, row2d, col2d)


def _dense1_body(x_ref, dT_ref, g1_ref, b1_ref, Wm_ref, bm_ref,
                 g2_ref, b2_ref, Wg_ref, xws_ref):
    x = x_ref[...]
    mu = jnp.mean(x, axis=0, keepdims=True)
    xc = x - mu
    var = jnp.mean(xc * xc, axis=0, keepdims=True)
    x = g1_ref[...] * xc * lax.rsqrt(var + 1e-5) + b1_ref[...]
    x = jnp.maximum(
        jnp.dot(x, Wm_ref[...], preferred_element_type=jnp.float32)
        + bm_ref[...], 0.0)
    mu2 = jnp.mean(x, axis=0, keepdims=True)
    xc2 = x - mu2
    var2 = jnp.mean(xc2 * xc2, axis=0, keepdims=True)
    x = g2_ref[...] * xc2 * lax.rsqrt(var2 + 1e-5) + b2_ref[...]
    xw = jnp.dot(x, Wg_ref[...], preferred_element_type=jnp.float32)
    deg = dT_ref[:, 0:1] + dT_ref[:, 1:2] + 1.0
    xws_ref[...] = lax.rsqrt(deg) * xw


def _combine_body(n, p_ref, xws_ref, dT_ref, bg_ref, Wc_ref, bc_ref, out_ref):
    tot = p_ref[0, 0:n, :] + p_ref[1, 0:n, :] + xws_ref[...]
    deg = dT_ref[:, 0:1] + dT_ref[:, 1:2] + 1.0
    h = jnp.maximum(lax.rsqrt(deg) * tot + bg_ref[...], 0.0)
    logits = jnp.dot(h, Wc_ref[...], preferred_element_type=jnp.float32) \
        + bc_ref[...]
    m = jnp.max(logits, axis=1, keepdims=True)
    lse = jnp.log(jnp.sum(jnp.exp(logits - m), axis=1, keepdims=True))
    out_ref[...] = logits - m - lse


def kernel(high_dim_features, low_dim_features, edge_index,
           g1, b1, Wm, bm, g2, b2, Wg, bg, Wc, bc):
    del low_dim_features
    n, d = high_dim_features.shape
    e = edge_index.shape[1]
    cpt = -(-e // (NW * CHUNK))
    cpt = -(-cpt // SB) * SB
    e_pad = NW * cpt * CHUNK

    pad_ar = jnp.arange(e_pad - e, dtype=jnp.int32)
    row = jnp.concatenate(
        [edge_index[0], pad_ar % n]).reshape(-1, CHUNK)
    col = jnp.concatenate(
        [edge_index[1], n + pad_ar % (NP - n)]).reshape(-1, CHUNK)

    degs = _deg_call(col, cpt)
    degsT = jnp.swapaxes(degs, 0, 1)[:n]

    xws = pl.pallas_call(
        _dense1_body,
        out_shape=jax.ShapeDtypeStruct((n, d), jnp.float32),
    )(high_dim_features, degsT,
      g1.reshape(1, -1), b1.reshape(1, -1), Wm, bm.reshape(1, -1),
      g2.reshape(1, -1), b2.reshape(1, -1), Wg)

    parts = _gs_call(xws, row, col, cpt)

    out = pl.pallas_call(
        functools.partial(_combine_body, n),
        out_shape=jax.ShapeDtypeStruct((n, Wc.shape[1]), jnp.float32),
    )(parts, xws, degsT, bg.reshape(1, -1), Wc, bc.reshape(1, -1))
    return out


# speedup vs baseline: 35.3659x; 1.0171x over previous
import functools

import jax
import jax.numpy as jnp
from jax import lax
from jax.experimental import pallas as pl
from jax.experimental.pallas import tpu as pltpu
from jax.experimental.pallas import tpu_sc as plsc

NC = 2
NS = 16
NW = NC * NS
CHUNK = 128
SB = 16
NP = 10240


def _deg_body(cpt, col_hbm, out_hbm, col_v, ones_v, deg_sh, sem):
    del sem
    c = lax.axis_index("c")
    s = lax.axis_index("s")
    w = s * NC + c
    rpt = NP // NS

    for j in range(8):
        ones_v[pl.ds(16 * j, 16)] = jnp.zeros((16,), jnp.float32)
    for j in range(rpt // CHUNK):
        pltpu.sync_copy(ones_v, deg_sh.at[pl.ds(s * rpt + j * CHUNK, CHUNK)])

    pltpu.sync_copy(col_hbm.at[pl.ds(w * cpt, cpt)], col_v)
    for j in range(8):
        ones_v[pl.ds(16 * j, 16)] = jnp.ones((16,), jnp.float32)
    plsc.subcore_barrier()

    def body(j, carry):
        pltpu.sync_copy(ones_v, deg_sh.at[col_v.at[j]], add=True)
        return carry

    lax.fori_loop(0, cpt, body, 0)
    plsc.subcore_barrier()
    pltpu.sync_copy(deg_sh.at[pl.ds(s * rpt, rpt)],
                    out_hbm.at[c, pl.ds(s * rpt, rpt)])


def _deg_call(col2d, cpt):
    kfn = pl.kernel(
        functools.partial(_deg_body, cpt),
        out_type=jax.ShapeDtypeStruct((NC, NP), jnp.float32),
        mesh=plsc.VectorSubcoreMesh(core_axis_name="c", subcore_axis_name="s"),
        scratch_types=[
            pltpu.VMEM((cpt, CHUNK), jnp.int32),
            pltpu.VMEM((CHUNK,), jnp.float32),
            pltpu.VMEM_SHARED((NP,), jnp.float32),
            pltpu.SemaphoreType.DMA,
        ],
    )
    return kfn(col2d)


def _gs_body(cpt, xws_hbm, row_hbm, col_hbm, out_hbm,
             row_v, col_v, rows_v0, rows_v1, zb, acc_sh, sem0, sem1):
    c = lax.axis_index("c")
    s = lax.axis_index("s")
    w = s * NC + c
    rpt = NP // NS
    bufs = (rows_v0, rows_v1)
    sems = (sem0, sem1)

    for i in range(8):
        for j in range(8):
            zb[i, pl.ds(16 * j, 16)] = jnp.zeros((16,), jnp.float32)
    for j in range(rpt // 8):
        pltpu.sync_copy(zb, acc_sh.at[pl.ds(s * rpt + j * 8, 8)])
    plsc.subcore_barrier()

    def stage(t, carry):
        base = w * cpt + t * SB
        pltpu.sync_copy(row_hbm.at[pl.ds(base, SB)], row_v)
        pltpu.sync_copy(col_hbm.at[pl.ds(base, SB)], col_v)
        pltpu.async_copy(xws_hbm.at[row_v.at[0]], bufs[0], sems[0])

        def body(i, carry2):
            for b in range(2):
                j = 2 * i + b

                @pl.when(j + 1 < SB)
                def _():
                    pltpu.async_copy(xws_hbm.at[row_v.at[j + 1]],
                                     bufs[1 - b], sems[1 - b])

                pltpu.make_async_copy(xws_hbm.at[row_v.at[j]],
                                      bufs[b], sems[b]).wait()
                pltpu.sync_copy(bufs[b], acc_sh.at[col_v.at[j]], add=True)
            return carry2

        lax.fori_loop(0, SB // 2, body, 0)
        return carry

    lax.fori_loop(0, cpt // SB, stage, 0)
    plsc.subcore_barrier()
    pltpu.sync_copy(acc_sh.at[pl.ds(s * rpt, rpt)],
                    out_hbm.at[c, pl.ds(s * rpt, rpt)])


def _gs_call(xws, row2d, col2d, cpt):
    kfn = pl.kernel(
        functools.partial(_gs_body, cpt),
        out_type=jax.ShapeDtypeStruct((NC, NP, 128), jnp.float32),
        mesh=plsc.VectorSubcoreMesh(core_axis_name="c", subcore_axis_name="s"),
        scratch_types=[
            pltpu.VMEM((SB, CHUNK), jnp.int32),
            pltpu.VMEM((SB, CHUNK), jnp.int32),
            pltpu.VMEM((CHUNK, 128), jnp.float32),
            pltpu.VMEM((CHUNK, 128), jnp.float32),
            pltpu.VMEM((8, 128), jnp.float32),
            pltpu.VMEM_SHARED((NP, 128), jnp.float32),
            pltpu.SemaphoreType.DMA,
            pltpu.SemaphoreType.DMA,
        ],
    )
    return kfn(xws, row2d, col2d)


def _dense_body(x_ref, g1_ref, b1_ref, Wm_ref, bm_ref,
                g2_ref, b2_ref, Wg_ref, xw_ref):
    x = x_ref[...]
    mu = jnp.mean(x, axis=0, keepdims=True)
    xc = x - mu
    var = jnp.mean(xc * xc, axis=0, keepdims=True)
    x = g1_ref[...] * xc * lax.rsqrt(var + 1e-5) + b1_ref[...]
    x = jnp.maximum(
        jnp.dot(x, Wm_ref[...], preferred_element_type=jnp.float32)
        + bm_ref[...], 0.0)
    mu2 = jnp.mean(x, axis=0, keepdims=True)
    xc2 = x - mu2
    var2 = jnp.mean(xc2 * xc2, axis=0, keepdims=True)
    x = g2_ref[...] * xc2 * lax.rsqrt(var2 + 1e-5) + b2_ref[...]
    xw_ref[...] = jnp.dot(x, Wg_ref[...], preferred_element_type=jnp.float32)


def _scale_body(xw_ref, dT_ref, xws_ref):
    deg = dT_ref[:, 0:1] + dT_ref[:, 1:2] + 1.0
    xws_ref[...] = lax.rsqrt(deg) * xw_ref[...]


def _combine_body(n, p_ref, xws_ref, dT_ref, bg_ref, Wc_ref, bc_ref,
                  out_ref):
    tot = p_ref[0, 0:n, :] + p_ref[1, 0:n, :] + xws_ref[...]
    deg = dT_ref[:, 0:1] + dT_ref[:, 1:2] + 1.0
    h = jnp.maximum(lax.rsqrt(deg) * tot + bg_ref[...], 0.0)
    logits = jnp.dot(h, Wc_ref[...], preferred_element_type=jnp.float32) \
        + bc_ref[...]
    m = jnp.max(logits, axis=1, keepdims=True)
    lse = jnp.log(jnp.sum(jnp.exp(logits - m), axis=1, keepdims=True))
    out_ref[...] = logits - m - lse


def kernel(high_dim_features, low_dim_features, edge_index,
           g1, b1, Wm, bm, g2, b2, Wg, bg, Wc, bc):
    del low_dim_features
    n, d = high_dim_features.shape
    e = edge_index.shape[1]
    cpt = -(-e // (NW * CHUNK))
    cpt = -(-cpt // SB) * SB
    e_pad = NW * cpt * CHUNK

    pad_ar = jnp.arange(e_pad - e, dtype=jnp.int32)
    row = jnp.concatenate(
        [edge_index[0], pad_ar % n]).reshape(-1, CHUNK)
    col = jnp.concatenate(
        [edge_index[1], n + pad_ar % (NP - n)]).reshape(-1, CHUNK)

    degs = _deg_call(col, cpt)
    degsT = jnp.swapaxes(degs, 0, 1)[:n]

    xw = pl.pallas_call(
        _dense_body,
        out_shape=jax.ShapeDtypeStruct((n, d), jnp.float32),
    )(high_dim_features,
      g1.reshape(1, -1), b1.reshape(1, -1), Wm, bm.reshape(1, -1),
      g2.reshape(1, -1), b2.reshape(1, -1), Wg)

    xws = pl.pallas_call(
        _scale_body,
        out_shape=jax.ShapeDtypeStruct((n, d), jnp.float32),
    )(xw, degsT)

    parts = _gs_call(xws, row, col, cpt)

    out = pl.pallas_call(
        functools.partial(_combine_body, n),
        out_shape=jax.ShapeDtypeStruct((n, Wc.shape[1]), jnp.float32),
    )(parts, xws, degsT, bg.reshape(1, -1), Wc, bc.reshape(1, -1))
    return out


# speedup vs baseline: 36.6156x; 1.0353x over previous
import functools

import jax
import jax.numpy as jnp
from jax import lax
from jax.experimental import pallas as pl
from jax.experimental.pallas import tpu as pltpu
from jax.experimental.pallas import tpu_sc as plsc

NC = 2
NS = 16
NW = NC * NS
CHUNK = 128
SB = 40
NP = 10240


def _deg_body(cpt, col_hbm, out_hbm, col_v, ones_v, deg_sh, sem):
    del sem
    c = lax.axis_index("c")
    s = lax.axis_index("s")
    w = s * NC + c
    rpt = NP // NS

    for j in range(8):
        ones_v[pl.ds(16 * j, 16)] = jnp.zeros((16,), jnp.float32)
    for j in range(rpt // CHUNK):
        pltpu.sync_copy(ones_v, deg_sh.at[pl.ds(s * rpt + j * CHUNK, CHUNK)])

    pltpu.sync_copy(col_hbm.at[pl.ds(w * cpt, cpt)], col_v)
    for j in range(8):
        ones_v[pl.ds(16 * j, 16)] = jnp.ones((16,), jnp.float32)
    plsc.subcore_barrier()

    def body(j, carry):
        pltpu.sync_copy(ones_v, deg_sh.at[col_v.at[j]], add=True)
        return carry

    lax.fori_loop(0, cpt, body, 0)
    plsc.subcore_barrier()
    pltpu.sync_copy(deg_sh.at[pl.ds(s * rpt, rpt)],
                    out_hbm.at[c, pl.ds(s * rpt, rpt)])


def _deg_call(col2d, cpt):
    kfn = pl.kernel(
        functools.partial(_deg_body, cpt),
        out_type=jax.ShapeDtypeStruct((NC, NP), jnp.float32),
        mesh=plsc.VectorSubcoreMesh(core_axis_name="c", subcore_axis_name="s"),
        scratch_types=[
            pltpu.VMEM((cpt, CHUNK), jnp.int32),
            pltpu.VMEM((CHUNK,), jnp.float32),
            pltpu.VMEM_SHARED((NP,), jnp.float32),
            pltpu.SemaphoreType.DMA,
        ],
    )
    return kfn(col2d)


def _gs_body(cpt, xws_hbm, row_hbm, col_hbm, out_hbm,
             row_v, col_v, rows_v0, rows_v1, zb, acc_sh, sem0, sem1):
    c = lax.axis_index("c")
    s = lax.axis_index("s")
    w = s * NC + c
    rpt = NP // NS
    bufs = (rows_v0, rows_v1)
    sems = (sem0, sem1)

    for i in range(8):
        for j in range(8):
            zb[i, pl.ds(16 * j, 16)] = jnp.zeros((16,), jnp.float32)
    for j in range(rpt // 8):
        pltpu.sync_copy(zb, acc_sh.at[pl.ds(s * rpt + j * 8, 8)])
    plsc.subcore_barrier()

    def stage(t, carry):
        base = w * cpt + t * SB
        pltpu.sync_copy(row_hbm.at[pl.ds(base, SB)], row_v)
        pltpu.sync_copy(col_hbm.at[pl.ds(base, SB)], col_v)
        pltpu.async_copy(xws_hbm.at[row_v.at[0]], bufs[0], sems[0])

        def body(i, carry2):
            for b in range(2):
                j = 2 * i + b

                @pl.when(j + 1 < SB)
                def _():
                    pltpu.async_copy(xws_hbm.at[row_v.at[j + 1]],
                                     bufs[1 - b], sems[1 - b])

                pltpu.make_async_copy(xws_hbm.at[row_v.at[j]],
                                      bufs[b], sems[b]).wait()
                pltpu.sync_copy(bufs[b], acc_sh.at[col_v.at[j]], add=True)
            return carry2

        lax.fori_loop(0, SB // 2, body, 0)
        return carry

    lax.fori_loop(0, cpt // SB, stage, 0)
    plsc.subcore_barrier()
    pltpu.sync_copy(acc_sh.at[pl.ds(s * rpt, rpt)],
                    out_hbm.at[c, pl.ds(s * rpt, rpt)])


def _gs_call(xws, row2d, col2d, cpt):
    kfn = pl.kernel(
        functools.partial(_gs_body, cpt),
        out_type=jax.ShapeDtypeStruct((NC, NP, 128), jnp.float32),
        mesh=plsc.VectorSubcoreMesh(core_axis_name="c", subcore_axis_name="s"),
        scratch_types=[
            pltpu.VMEM((SB, CHUNK), jnp.int32),
            pltpu.VMEM((SB, CHUNK), jnp.int32),
            pltpu.VMEM((CHUNK, 128), jnp.float32),
            pltpu.VMEM((CHUNK, 128), jnp.float32),
            pltpu.VMEM((8, 128), jnp.float32),
            pltpu.VMEM_SHARED((NP, 128), jnp.float32),
            pltpu.SemaphoreType.DMA,
            pltpu.SemaphoreType.DMA,
        ],
    )
    return kfn(xws, row2d, col2d)


def _pad_body(n, e, er_ref, ec_ref, row_ref, col_ref):
    del e
    npad = row_ref.shape[0] - er_ref.shape[0]
    fi = (lax.broadcasted_iota(jnp.int32, (npad, CHUNK), 0) * CHUNK
          + lax.broadcasted_iota(jnp.int32, (npad, CHUNK), 1))
    row_ref[...] = jnp.concatenate([er_ref[...], fi % n], axis=0)
    col_ref[...] = jnp.concatenate([ec_ref[...], n + fi % (NP - n)], axis=0)


def _dense_body(x_ref, g1_ref, b1_ref, Wm_ref, bm_ref,
                g2_ref, b2_ref, Wg_ref, xw_ref):
    x = x_ref[...]
    mu = jnp.mean(x, axis=0, keepdims=True)
    xc = x - mu
    var = jnp.mean(xc * xc, axis=0, keepdims=True)
    x = g1_ref[...] * xc * lax.rsqrt(var + 1e-5) + b1_ref[...]
    x = jnp.maximum(
        jnp.dot(x, Wm_ref[...], preferred_element_type=jnp.float32)
        + bm_ref[...], 0.0)
    mu2 = jnp.mean(x, axis=0, keepdims=True)
    xc2 = x - mu2
    var2 = jnp.mean(xc2 * xc2, axis=0, keepdims=True)
    x = g2_ref[...] * xc2 * lax.rsqrt(var2 + 1e-5) + b2_ref[...]
    xw_ref[...] = jnp.dot(x, Wg_ref[...], preferred_element_type=jnp.float32)


def _scale_body(xw_ref, dT_ref, xws_ref):
    deg = dT_ref[:, 0:1] + dT_ref[:, 1:2] + 1.0
    xws_ref[...] = lax.rsqrt(deg) * xw_ref[...]


def _combine_body(n, p_ref, xws_ref, dT_ref, bg_ref, Wc_ref, bc_ref,
                  out_ref):
    tot = p_ref[0, 0:n, :] + p_ref[1, 0:n, :] + xws_ref[...]
    deg = dT_ref[:, 0:1] + dT_ref[:, 1:2] + 1.0
    h = jnp.maximum(lax.rsqrt(deg) * tot + bg_ref[...], 0.0)
    logits = jnp.dot(h, Wc_ref[...], preferred_element_type=jnp.float32) \
        + bc_ref[...]
    m = jnp.max(logits, axis=1, keepdims=True)
    lse = jnp.log(jnp.sum(jnp.exp(logits - m), axis=1, keepdims=True))
    out_ref[...] = logits - m - lse


def kernel(high_dim_features, low_dim_features, edge_index,
           g1, b1, Wm, bm, g2, b2, Wg, bg, Wc, bc):
    del low_dim_features
    n, d = high_dim_features.shape
    e = edge_index.shape[1]
    cpt = -(-e // (NW * CHUNK))
    cpt = -(-cpt // SB) * SB
    e_pad = NW * cpt * CHUNK

    if e % CHUNK == 0 and e_pad > e:
        row, col = pl.pallas_call(
            functools.partial(_pad_body, n, e),
            out_shape=[
                jax.ShapeDtypeStruct((e_pad // CHUNK, CHUNK), jnp.int32),
                jax.ShapeDtypeStruct((e_pad // CHUNK, CHUNK), jnp.int32)],
        )(edge_index[0].reshape(-1, CHUNK), edge_index[1].reshape(-1, CHUNK))
    else:
        pad_ar = jnp.arange(e_pad - e, dtype=jnp.int32)
        row = jnp.concatenate(
            [edge_index[0], pad_ar % n]).reshape(-1, CHUNK)
        col = jnp.concatenate(
            [edge_index[1], n + pad_ar % (NP - n)]).reshape(-1, CHUNK)

    degs = _deg_call(col, cpt)
    degsT = jnp.swapaxes(degs, 0, 1)[:n]

    xw = pl.pallas_call(
        _dense_body,
        out_shape=jax.ShapeDtypeStruct((n, d), jnp.float32),
    )(high_dim_features,
      g1.reshape(1, -1), b1.reshape(1, -1), Wm, bm.reshape(1, -1),
      g2.reshape(1, -1), b2.reshape(1, -1), Wg)

    xws = pl.pallas_call(
        _scale_body,
        out_shape=jax.ShapeDtypeStruct((n, d), jnp.float32),
    )(xw, degsT)

    parts = _gs_call(xws, row, col, cpt)

    out = pl.pallas_call(
        functools.partial(_combine_body, n),
        out_shape=jax.ShapeDtypeStruct((n, Wc.shape[1]), jnp.float32),
    )(parts, xws, degsT, bg.reshape(1, -1), Wc, bc.reshape(1, -1))
    return out


# speedup vs baseline: 36.6296x; 1.0004x over previous
import functools

import jax
import jax.numpy as jnp
from jax import lax
from jax.experimental import pallas as pl
from jax.experimental.pallas import tpu as pltpu
from jax.experimental.pallas import tpu_sc as plsc

NC = 2
NS = 16
NW = NC * NS
CHUNK = 128
SB = 40
NP = 10240


def _deg_body(cpt, col_hbm, out_hbm, col_v, ones_v, deg_sh, sem):
    del sem
    c = lax.axis_index("c")
    s = lax.axis_index("s")
    w = s * NC + c
    rpt = NP // NS

    for j in range(8):
        ones_v[pl.ds(16 * j, 16)] = jnp.zeros((16,), jnp.float32)
    for j in range(rpt // CHUNK):
        pltpu.sync_copy(ones_v, deg_sh.at[pl.ds(s * rpt + j * CHUNK, CHUNK)])

    pltpu.sync_copy(col_hbm.at[pl.ds(w * cpt, cpt)], col_v)
    for j in range(8):
        ones_v[pl.ds(16 * j, 16)] = jnp.ones((16,), jnp.float32)
    plsc.subcore_barrier()

    def body(j, carry):
        pltpu.sync_copy(ones_v, deg_sh.at[col_v.at[j]], add=True)
        return carry

    lax.fori_loop(0, cpt, body, 0)
    plsc.subcore_barrier()
    pltpu.sync_copy(deg_sh.at[pl.ds(s * rpt, rpt)],
                    out_hbm.at[c, pl.ds(s * rpt, rpt)])


def _deg_call(col2d, cpt):
    kfn = pl.kernel(
        functools.partial(_deg_body, cpt),
        out_type=jax.ShapeDtypeStruct((NC, NP), jnp.float32),
        mesh=plsc.VectorSubcoreMesh(core_axis_name="c", subcore_axis_name="s"),
        scratch_types=[
            pltpu.VMEM((cpt, CHUNK), jnp.int32),
            pltpu.VMEM((CHUNK,), jnp.float32),
            pltpu.VMEM_SHARED((NP,), jnp.float32),
            pltpu.SemaphoreType.DMA,
        ],
    )
    return kfn(col2d)


def _gs_body(cpt, xws_hbm, row_hbm, col_hbm, out_hbm,
             row_v, col_v, rows_v0, rows_v1, zb, acc_sh,
             gsem0, gsem1, ssem0, ssem1):
    c = lax.axis_index("c")
    s = lax.axis_index("s")
    w = s * NC + c
    rpt = NP // NS
    bufs = (rows_v0, rows_v1)
    gsems = (gsem0, gsem1)
    ssems = (ssem0, ssem1)

    for i in range(8):
        for j in range(8):
            zb[i, pl.ds(16 * j, 16)] = jnp.zeros((16,), jnp.float32)
    for j in range(rpt // 8):
        pltpu.sync_copy(zb, acc_sh.at[pl.ds(s * rpt + j * 8, 8)])
    plsc.subcore_barrier()

    def stage(t, carry):
        base = w * cpt + t * SB
        pltpu.sync_copy(row_hbm.at[pl.ds(base, SB)], row_v)
        pltpu.sync_copy(col_hbm.at[pl.ds(base, SB)], col_v)
        pltpu.async_copy(xws_hbm.at[row_v.at[0]], bufs[0], gsems[0])

        def body(i, carry2):
            for b in range(2):
                j = 2 * i + b

                @pl.when(j + 1 < SB)
                def _():
                    @pl.when(j >= 1)
                    def _():
                        pltpu.make_async_copy(
                            bufs[1 - b], acc_sh.at[col_v.at[j]],
                            ssems[1 - b]).wait()

                    pltpu.async_copy(xws_hbm.at[row_v.at[j + 1]],
                                     bufs[1 - b], gsems[1 - b])

                pltpu.make_async_copy(xws_hbm.at[row_v.at[j]],
                                      bufs[b], gsems[b]).wait()
                pltpu.async_copy(bufs[b], acc_sh.at[col_v.at[j]],
                                 ssems[b], add=True)
            return carry2

        lax.fori_loop(0, SB // 2, body, 0)
        pltpu.make_async_copy(bufs[0], acc_sh.at[col_v.at[0]],
                              ssems[0]).wait()
        pltpu.make_async_copy(bufs[1], acc_sh.at[col_v.at[0]],
                              ssems[1]).wait()
        return carry

    lax.fori_loop(0, cpt // SB, stage, 0)
    plsc.subcore_barrier()
    pltpu.sync_copy(acc_sh.at[pl.ds(s * rpt, rpt)],
                    out_hbm.at[c, pl.ds(s * rpt, rpt)])


def _gs_call(xws, row2d, col2d, cpt):
    kfn = pl.kernel(
        functools.partial(_gs_body, cpt),
        out_type=jax.ShapeDtypeStruct((NC, NP, 128), jnp.float32),
        mesh=plsc.VectorSubcoreMesh(core_axis_name="c", subcore_axis_name="s"),
        scratch_types=[
            pltpu.VMEM((SB, CHUNK), jnp.int32),
            pltpu.VMEM((SB, CHUNK), jnp.int32),
            pltpu.VMEM((CHUNK, 128), jnp.float32),
            pltpu.VMEM((CHUNK, 128), jnp.float32),
            pltpu.VMEM((8, 128), jnp.float32),
            pltpu.VMEM_SHARED((NP, 128), jnp.float32),
            pltpu.SemaphoreType.DMA,
            pltpu.SemaphoreType.DMA,
            pltpu.SemaphoreType.DMA,
            pltpu.SemaphoreType.DMA,
        ],
    )
    return kfn(xws, row2d, col2d)


def _pad_body(n, e, er_ref, ec_ref, row_ref, col_ref):
    del e
    npad = row_ref.shape[0] - er_ref.shape[0]
    fi = (lax.broadcasted_iota(jnp.int32, (npad, CHUNK), 0) * CHUNK
          + lax.broadcasted_iota(jnp.int32, (npad, CHUNK), 1))
    row_ref[...] = jnp.concatenate([er_ref[...], fi % n], axis=0)
    col_ref[...] = jnp.concatenate([ec_ref[...], n + fi % (NP - n)], axis=0)


def _dense_body(x_ref, g1_ref, b1_ref, Wm_ref, bm_ref,
                g2_ref, b2_ref, Wg_ref, xw_ref):
    x = x_ref[...]
    mu = jnp.mean(x, axis=0, keepdims=True)
    xc = x - mu
    var = jnp.mean(xc * xc, axis=0, keepdims=True)
    x = g1_ref[...] * xc * lax.rsqrt(var + 1e-5) + b1_ref[...]
    x = jnp.maximum(
        jnp.dot(x, Wm_ref[...], preferred_element_type=jnp.float32)
        + bm_ref[...], 0.0)
    mu2 = jnp.mean(x, axis=0, keepdims=True)
    xc2 = x - mu2
    var2 = jnp.mean(xc2 * xc2, axis=0, keepdims=True)
    x = g2_ref[...] * xc2 * lax.rsqrt(var2 + 1e-5) + b2_ref[...]
    xw_ref[...] = jnp.dot(x, Wg_ref[...], preferred_element_type=jnp.float32)


def _scale_body(xw_ref, dT_ref, xws_ref):
    deg = dT_ref[:, 0:1] + dT_ref[:, 1:2] + 1.0
    xws_ref[...] = lax.rsqrt(deg) * xw_ref[...]


def _combine_body(n, p_ref, xws_ref, dT_ref, bg_ref, Wc_ref, bc_ref,
                  out_ref):
    tot = p_ref[0, 0:n, :] + p_ref[1, 0:n, :] + xws_ref[...]
    deg = dT_ref[:, 0:1] + dT_ref[:, 1:2] + 1.0
    h = jnp.maximum(lax.rsqrt(deg) * tot + bg_ref[...], 0.0)
    logits = jnp.dot(h, Wc_ref[...], preferred_element_type=jnp.float32) \
        + bc_ref[...]
    m = jnp.max(logits, axis=1, keepdims=True)
    lse = jnp.log(jnp.sum(jnp.exp(logits - m), axis=1, keepdims=True))
    out_ref[...] = logits - m - lse


def kernel(high_dim_features, low_dim_features, edge_index,
           g1, b1, Wm, bm, g2, b2, Wg, bg, Wc, bc):
    del low_dim_features
    n, d = high_dim_features.shape
    e = edge_index.shape[1]
    cpt = -(-e // (NW * CHUNK))
    cpt = -(-cpt // SB) * SB
    e_pad = NW * cpt * CHUNK

    if e % CHUNK == 0 and e_pad > e:
        row, col = pl.pallas_call(
            functools.partial(_pad_body, n, e),
            out_shape=[
                jax.ShapeDtypeStruct((e_pad // CHUNK, CHUNK), jnp.int32),
                jax.ShapeDtypeStruct((e_pad // CHUNK, CHUNK), jnp.int32)],
        )(edge_index[0].reshape(-1, CHUNK), edge_index[1].reshape(-1, CHUNK))
    else:
        pad_ar = jnp.arange(e_pad - e, dtype=jnp.int32)
        row = jnp.concatenate(
            [edge_index[0], pad_ar % n]).reshape(-1, CHUNK)
        col = jnp.concatenate(
            [edge_index[1], n + pad_ar % (NP - n)]).reshape(-1, CHUNK)

    degs = _deg_call(col, cpt)
    degsT = jnp.swapaxes(degs, 0, 1)[:n]

    xw = pl.pallas_call(
        _dense_body,
        out_shape=jax.ShapeDtypeStruct((n, d), jnp.float32),
    )(high_dim_features,
      g1.reshape(1, -1), b1.reshape(1, -1), Wm, bm.reshape(1, -1),
      g2.reshape(1, -1), b2.reshape(1, -1), Wg)

    xws = pl.pallas_call(
        _scale_body,
        out_shape=jax.ShapeDtypeStruct((n, d), jnp.float32),
    )(xw, degsT)

    parts = _gs_call(xws, row, col, cpt)

    out = pl.pallas_call(
        functools.partial(_combine_body, n),
        out_shape=jax.ShapeDtypeStruct((n, Wc.shape[1]), jnp.float32),
    )(parts, xws, degsT, bg.reshape(1, -1), Wc, bc.reshape(1, -1))
    return out


# speedup vs baseline: 39.6931x; 1.0836x over previous
import functools

import jax
import jax.numpy as jnp
from jax import lax
from jax.experimental import pallas as pl
from jax.experimental.pallas import tpu as pltpu
from jax.experimental.pallas import tpu_sc as plsc

NC = 2
NS = 16
NW = NC * NS
CHUNK = 128
SB = 40
NP = 10240


def _deg_body(cpt, col_hbm, out_hbm, col_v, ones_v, deg_sh, sem):
    del sem
    c = lax.axis_index("c")
    s = lax.axis_index("s")
    w = s * NC + c
    rpt = NP // NS

    for j in range(8):
        ones_v[pl.ds(16 * j, 16)] = jnp.zeros((16,), jnp.float32)
    for j in range(rpt // CHUNK):
        pltpu.sync_copy(ones_v, deg_sh.at[pl.ds(s * rpt + j * CHUNK, CHUNK)])

    pltpu.sync_copy(col_hbm.at[pl.ds(w * cpt, cpt)], col_v)
    for j in range(8):
        ones_v[pl.ds(16 * j, 16)] = jnp.ones((16,), jnp.float32)
    plsc.subcore_barrier()

    def body(j, carry):
        pltpu.sync_copy(ones_v, deg_sh.at[col_v.at[j]], add=True)
        return carry

    lax.fori_loop(0, cpt, body, 0)
    plsc.subcore_barrier()
    pltpu.sync_copy(deg_sh.at[pl.ds(s * rpt, rpt)],
                    out_hbm.at[c, pl.ds(s * rpt, rpt)])


def _deg_call(col2d, cpt):
    kfn = pl.kernel(
        functools.partial(_deg_body, cpt),
        out_type=jax.ShapeDtypeStruct((NC, NP), jnp.float32),
        mesh=plsc.VectorSubcoreMesh(core_axis_name="c", subcore_axis_name="s"),
        scratch_types=[
            pltpu.VMEM((cpt, CHUNK), jnp.int32),
            pltpu.VMEM((CHUNK,), jnp.float32),
            pltpu.VMEM_SHARED((NP,), jnp.float32),
            pltpu.SemaphoreType.DMA,
        ],
    )
    return kfn(col2d)


def _gs_body(cpt, xws_hbm, row_hbm, col_hbm, out_hbm,
             row_v, col_v, rows_v0, rows_v1, zb, acc_sh,
             gsem0, gsem1, ssem0, ssem1):
    c = lax.axis_index("c")
    s = lax.axis_index("s")
    w = s * NC + c
    rpt = NP // NS
    bufs = (rows_v0, rows_v1)
    gsems = (gsem0, gsem1)
    ssems = (ssem0, ssem1)

    for i in range(8):
        for j in range(8):
            zb[i, pl.ds(16 * j, 16)] = jnp.zeros((16,), jnp.float32)
    for j in range(rpt // 8):
        pltpu.sync_copy(zb, acc_sh.at[pl.ds(s * rpt + j * 8, 8)])
    plsc.subcore_barrier()

    def stage(t, carry):
        base = w * cpt + t * SB
        pltpu.sync_copy(row_hbm.at[pl.ds(base, SB)], row_v)
        pltpu.sync_copy(col_hbm.at[pl.ds(base, SB)], col_v)
        pltpu.async_copy(xws_hbm.at[row_v.at[0]], bufs[0], gsems[0])

        def body(i, carry2):
            for b in range(2):
                j = 2 * i + b

                @pl.when(j + 1 < SB)
                def _():
                    @pl.when(j >= 1)
                    def _():
                        pltpu.make_async_copy(
                            bufs[1 - b], acc_sh.at[col_v.at[j]],
                            ssems[1 - b]).wait()

                    pltpu.async_copy(xws_hbm.at[row_v.at[j + 1]],
                                     bufs[1 - b], gsems[1 - b])

                pltpu.make_async_copy(xws_hbm.at[row_v.at[j]],
                                      bufs[b], gsems[b]).wait()
                pltpu.async_copy(bufs[b], acc_sh.at[col_v.at[j]],
                                 ssems[b], add=True)
            return carry2

        lax.fori_loop(0, SB // 2, body, 0)
        pltpu.make_async_copy(bufs[0], acc_sh.at[col_v.at[0]],
                              ssems[0]).wait()
        pltpu.make_async_copy(bufs[1], acc_sh.at[col_v.at[0]],
                              ssems[1]).wait()
        return carry

    lax.fori_loop(0, cpt // SB, stage, 0)
    plsc.subcore_barrier()
    pltpu.sync_copy(acc_sh.at[pl.ds(s * rpt, rpt)],
                    out_hbm.at[c, pl.ds(s * rpt, rpt)])


def _gs_call(xws, row2d, col2d, cpt):
    kfn = pl.kernel(
        functools.partial(_gs_body, cpt),
        out_type=jax.ShapeDtypeStruct((NC, NP, 128), jnp.float32),
        mesh=plsc.VectorSubcoreMesh(core_axis_name="c", subcore_axis_name="s"),
        scratch_types=[
            pltpu.VMEM((SB, CHUNK), jnp.int32),
            pltpu.VMEM((SB, CHUNK), jnp.int32),
            pltpu.VMEM((CHUNK, 128), jnp.float32),
            pltpu.VMEM((CHUNK, 128), jnp.float32),
            pltpu.VMEM((8, 128), jnp.float32),
            pltpu.VMEM_SHARED((NP, 128), jnp.float32),
            pltpu.SemaphoreType.DMA,
            pltpu.SemaphoreType.DMA,
            pltpu.SemaphoreType.DMA,
            pltpu.SemaphoreType.DMA,
        ],
    )
    return kfn(xws, row2d, col2d)


def _pad_body(n, e, ei_ref, row_ref, col_ref):
    nrows = e // CHUNK
    npad = row_ref.shape[0] - nrows
    er = ei_ref[0:1, :].reshape(nrows, CHUNK)
    ec = ei_ref[1:2, :].reshape(nrows, CHUNK)
    fi = (lax.broadcasted_iota(jnp.int32, (npad, CHUNK), 0) * CHUNK
          + lax.broadcasted_iota(jnp.int32, (npad, CHUNK), 1))
    row_ref[...] = jnp.concatenate([er, fi % n], axis=0)
    col_ref[...] = jnp.concatenate([ec, n + fi % (NP - n)], axis=0)


def _dense_body(x_ref, g1_ref, b1_ref, Wm_ref, bm_ref,
                g2_ref, b2_ref, Wg_ref, xw_ref):
    x = x_ref[...]
    mu = jnp.mean(x, axis=0, keepdims=True)
    xc = x - mu
    var = jnp.mean(xc * xc, axis=0, keepdims=True)
    x = g1_ref[...] * xc * lax.rsqrt(var + 1e-5) + b1_ref[...]
    x = jnp.maximum(
        jnp.dot(x, Wm_ref[...], preferred_element_type=jnp.float32)
        + bm_ref[...], 0.0)
    mu2 = jnp.mean(x, axis=0, keepdims=True)
    xc2 = x - mu2
    var2 = jnp.mean(xc2 * xc2, axis=0, keepdims=True)
    x = g2_ref[...] * xc2 * lax.rsqrt(var2 + 1e-5) + b2_ref[...]
    xw_ref[...] = jnp.dot(x, Wg_ref[...], preferred_element_type=jnp.float32)


def _scale_body(xw_ref, dT_ref, xws_ref):
    deg = dT_ref[:, 0:1] + dT_ref[:, 1:2] + 1.0
    xws_ref[...] = lax.rsqrt(deg) * xw_ref[...]


def _combine_body(n, p_ref, xws_ref, dT_ref, bg_ref, Wc_ref, bc_ref,
                  out_ref):
    tot = p_ref[0, 0:n, :] + p_ref[1, 0:n, :] + xws_ref[...]
    deg = dT_ref[:, 0:1] + dT_ref[:, 1:2] + 1.0
    h = jnp.maximum(lax.rsqrt(deg) * tot + bg_ref[...], 0.0)
    logits = jnp.dot(h, Wc_ref[...], preferred_element_type=jnp.float32) \
        + bc_ref[...]
    m = jnp.max(logits, axis=1, keepdims=True)
    lse = jnp.log(jnp.sum(jnp.exp(logits - m), axis=1, keepdims=True))
    out_ref[...] = logits - m - lse


def kernel(high_dim_features, low_dim_features, edge_index,
           g1, b1, Wm, bm, g2, b2, Wg, bg, Wc, bc):
    del low_dim_features
    n, d = high_dim_features.shape
    e = edge_index.shape[1]
    cpt = -(-e // (NW * CHUNK))
    cpt = -(-cpt // SB) * SB
    e_pad = NW * cpt * CHUNK

    if e % CHUNK == 0 and e_pad > e:
        row, col = pl.pallas_call(
            functools.partial(_pad_body, n, e),
            out_shape=[
                jax.ShapeDtypeStruct((e_pad // CHUNK, CHUNK), jnp.int32),
                jax.ShapeDtypeStruct((e_pad // CHUNK, CHUNK), jnp.int32)],
        )(edge_index)
    else:
        pad_ar = jnp.arange(e_pad - e, dtype=jnp.int32)
        row = jnp.concatenate(
            [edge_index[0], pad_ar % n]).reshape(-1, CHUNK)
        col = jnp.concatenate(
            [edge_index[1], n + pad_ar % (NP - n)]).reshape(-1, CHUNK)

    degs = _deg_call(col, cpt)
    degsT = jnp.swapaxes(degs, 0, 1)[:n]

    xw = pl.pallas_call(
        _dense_body,
        out_shape=jax.ShapeDtypeStruct((n, d), jnp.float32),
    )(high_dim_features,
      g1.reshape(1, -1), b1.reshape(1, -1), Wm, bm.reshape(1, -1),
      g2.reshape(1, -1), b2.reshape(1, -1), Wg)

    xws = pl.pallas_call(
        _scale_body,
        out_shape=jax.ShapeDtypeStruct((n, d), jnp.float32),
    )(xw, degsT)

    parts = _gs_call(xws, row, col, cpt)

    out = pl.pallas_call(
        functools.partial(_combine_body, n),
        out_shape=jax.ShapeDtypeStruct((n, Wc.shape[1]), jnp.float32),
    )(parts, xws, degsT, bg.reshape(1, -1), Wc, bc.reshape(1, -1))
    return out


# speedup vs baseline: 41.3486x; 1.0417x over previous
import functools

import jax
import jax.numpy as jnp
from jax import lax
from jax.experimental import pallas as pl
from jax.experimental.pallas import tpu as pltpu
from jax.experimental.pallas import tpu_sc as plsc

NC = 2
NS = 16
NW = NC * NS
CHUNK = 128
SB = 40
NP = 10240


def _deg_body(cpt, col_hbm, out_hbm, col_v, ones_v, deg_sh, sem):
    del sem
    c = lax.axis_index("c")
    s = lax.axis_index("s")
    w = s * NC + c
    rpt = NP // NS

    for j in range(8):
        ones_v[pl.ds(16 * j, 16)] = jnp.zeros((16,), jnp.float32)
    for j in range(rpt // CHUNK):
        pltpu.sync_copy(ones_v, deg_sh.at[pl.ds(s * rpt + j * CHUNK, CHUNK)])

    pltpu.sync_copy(col_hbm.at[pl.ds(w * cpt, cpt)], col_v)
    for j in range(8):
        ones_v[pl.ds(16 * j, 16)] = jnp.ones((16,), jnp.float32)
    plsc.subcore_barrier()

    def body(j, carry):
        pltpu.sync_copy(ones_v, deg_sh.at[col_v.at[j]], add=True)
        return carry

    lax.fori_loop(0, cpt, body, 0)
    plsc.subcore_barrier()
    pltpu.sync_copy(deg_sh.at[pl.ds(s * rpt, rpt)],
                    out_hbm.at[c, pl.ds(s * rpt, rpt)])


def _deg_call(col2d, cpt):
    kfn = pl.kernel(
        functools.partial(_deg_body, cpt),
        out_type=jax.ShapeDtypeStruct((NC, NP), jnp.float32),
        mesh=plsc.VectorSubcoreMesh(core_axis_name="c", subcore_axis_name="s"),
        scratch_types=[
            pltpu.VMEM((cpt, CHUNK), jnp.int32),
            pltpu.VMEM((CHUNK,), jnp.float32),
            pltpu.VMEM_SHARED((NP,), jnp.float32),
            pltpu.SemaphoreType.DMA,
        ],
    )
    return kfn(col2d)


def _gs_body(cpt, xws_hbm, row_hbm, col_hbm, out_hbm,
             row_v, col_v, rows_v0, rows_v1, zb, acc_sh,
             gsem0, gsem1, ssem0, ssem1):
    c = lax.axis_index("c")
    s = lax.axis_index("s")
    w = s * NC + c
    rpt = NP // NS
    bufs = (rows_v0, rows_v1)
    gsems = (gsem0, gsem1)
    ssems = (ssem0, ssem1)

    for i in range(8):
        for j in range(8):
            zb[i, pl.ds(16 * j, 16)] = jnp.zeros((16,), jnp.float32)
    for j in range(rpt // 8):
        pltpu.sync_copy(zb, acc_sh.at[pl.ds(s * rpt + j * 8, 8)])
    plsc.subcore_barrier()

    def stage(t, carry):
        base = w * cpt + t * SB
        pltpu.sync_copy(row_hbm.at[pl.ds(base, SB)], row_v)
        pltpu.sync_copy(col_hbm.at[pl.ds(base, SB)], col_v)
        pltpu.async_copy(xws_hbm.at[row_v.at[0]], bufs[0], gsems[0])

        def body(i, carry2):
            for b in range(2):
                j = 2 * i + b

                @pl.when(j + 1 < SB)
                def _():
                    @pl.when(j >= 1)
                    def _():
                        pltpu.make_async_copy(
                            bufs[1 - b], acc_sh.at[col_v.at[j]],
                            ssems[1 - b]).wait()

                    pltpu.async_copy(xws_hbm.at[row_v.at[j + 1]],
                                     bufs[1 - b], gsems[1 - b])

                pltpu.make_async_copy(xws_hbm.at[row_v.at[j]],
                                      bufs[b], gsems[b]).wait()
                pltpu.async_copy(bufs[b], acc_sh.at[col_v.at[j]],
                                 ssems[b], add=True)
            return carry2

        lax.fori_loop(0, SB // 2, body, 0)
        pltpu.make_async_copy(bufs[0], acc_sh.at[col_v.at[0]],
                              ssems[0]).wait()
        pltpu.make_async_copy(bufs[1], acc_sh.at[col_v.at[0]],
                              ssems[1]).wait()
        return carry

    lax.fori_loop(0, cpt // SB, stage, 0)
    plsc.subcore_barrier()
    pltpu.sync_copy(acc_sh.at[pl.ds(s * rpt, rpt)],
                    out_hbm.at[c, pl.ds(s * rpt, rpt)])


def _gs_call(xws, row2d, col2d, cpt):
    kfn = pl.kernel(
        functools.partial(_gs_body, cpt),
        out_type=jax.ShapeDtypeStruct((NC, NP, 128), jnp.float32),
        mesh=plsc.VectorSubcoreMesh(core_axis_name="c", subcore_axis_name="s"),
        scratch_types=[
            pltpu.VMEM((SB, CHUNK), jnp.int32),
            pltpu.VMEM((SB, CHUNK), jnp.int32),
            pltpu.VMEM((CHUNK, 128), jnp.float32),
            pltpu.VMEM((CHUNK, 128), jnp.float32),
            pltpu.VMEM((8, 128), jnp.float32),
            pltpu.VMEM_SHARED((NP, 128), jnp.float32),
            pltpu.SemaphoreType.DMA,
            pltpu.SemaphoreType.DMA,
            pltpu.SemaphoreType.DMA,
            pltpu.SemaphoreType.DMA,
        ],
    )
    return kfn(xws, row2d, col2d)


def _pad_body(n, e, ei_ref, row_ref, col_ref):
    nrows = e // CHUNK
    npad = row_ref.shape[0] - nrows
    er = ei_ref[0:1, :].reshape(nrows, CHUNK)
    ec = ei_ref[1:2, :].reshape(nrows, CHUNK)
    fi = (lax.broadcasted_iota(jnp.int32, (npad, CHUNK), 0) * CHUNK
          + lax.broadcasted_iota(jnp.int32, (npad, CHUNK), 1))
    row_ref[...] = jnp.concatenate([er, fi % n], axis=0)
    col_ref[...] = jnp.concatenate([ec, n + fi % (NP - n)], axis=0)


def _dense_body(x_ref, g1_ref, b1_ref, Wm_ref, bm_ref,
                g2_ref, b2_ref, Wg_ref, xw_ref):
    x = x_ref[...]
    mu = jnp.mean(x, axis=0, keepdims=True)
    xc = x - mu
    var = jnp.mean(xc * xc, axis=0, keepdims=True)
    x = g1_ref[...] * xc * lax.rsqrt(var + 1e-5) + b1_ref[...]
    x = jnp.maximum(
        jnp.dot(x, Wm_ref[...], preferred_element_type=jnp.float32)
        + bm_ref[...], 0.0)
    mu2 = jnp.mean(x, axis=0, keepdims=True)
    xc2 = x - mu2
    var2 = jnp.mean(xc2 * xc2, axis=0, keepdims=True)
    x = g2_ref[...] * xc2 * lax.rsqrt(var2 + 1e-5) + b2_ref[...]
    xw_ref[...] = jnp.dot(x, Wg_ref[...], preferred_element_type=jnp.float32)


def _dinv_col(degs_ref, n):
    deg = degs_ref[0:1, :] + degs_ref[1:2, :] + 1.0
    return jnp.transpose(lax.rsqrt(deg))[0:n]


def _scale_body(n, xw_ref, degs_ref, xws_ref):
    xws_ref[...] = _dinv_col(degs_ref, n) * xw_ref[...]


def _combine_body(n, p_ref, xws_ref, degs_ref, bg_ref, Wc_ref, bc_ref,
                  out_ref):
    tot = p_ref[0, 0:n, :] + p_ref[1, 0:n, :] + xws_ref[...]
    h = jnp.maximum(_dinv_col(degs_ref, n) * tot + bg_ref[...], 0.0)
    logits = jnp.dot(h, Wc_ref[...], preferred_element_type=jnp.float32) \
        + bc_ref[...]
    m = jnp.max(logits, axis=1, keepdims=True)
    lse = jnp.log(jnp.sum(jnp.exp(logits - m), axis=1, keepdims=True))
    out_ref[...] = logits - m - lse


def kernel(high_dim_features, low_dim_features, edge_index,
           g1, b1, Wm, bm, g2, b2, Wg, bg, Wc, bc):
    del low_dim_features
    n, d = high_dim_features.shape
    e = edge_index.shape[1]
    cpt = -(-e // (NW * CHUNK))
    cpt = -(-cpt // SB) * SB
    e_pad = NW * cpt * CHUNK

    if e % CHUNK == 0 and e_pad > e:
        row, col = pl.pallas_call(
            functools.partial(_pad_body, n, e),
            out_shape=[
                jax.ShapeDtypeStruct((e_pad // CHUNK, CHUNK), jnp.int32),
                jax.ShapeDtypeStruct((e_pad // CHUNK, CHUNK), jnp.int32)],
        )(edge_index)
    else:
        pad_ar = jnp.arange(e_pad - e, dtype=jnp.int32)
        row = jnp.concatenate(
            [edge_index[0], pad_ar % n]).reshape(-1, CHUNK)
        col = jnp.concatenate(
            [edge_index[1], n + pad_ar % (NP - n)]).reshape(-1, CHUNK)

    degs = _deg_call(col, cpt)

    xw = pl.pallas_call(
        _dense_body,
        out_shape=jax.ShapeDtypeStruct((n, d), jnp.float32),
    )(high_dim_features,
      g1.reshape(1, -1), b1.reshape(1, -1), Wm, bm.reshape(1, -1),
      g2.reshape(1, -1), b2.reshape(1, -1), Wg)

    xws = pl.pallas_call(
        functools.partial(_scale_body, n),
        out_shape=jax.ShapeDtypeStruct((n, d), jnp.float32),
    )(xw, degs)

    parts = _gs_call(xws, row, col, cpt)

    out = pl.pallas_call(
        functools.partial(_combine_body, n),
        out_shape=jax.ShapeDtypeStruct((n, Wc.shape[1]), jnp.float32),
    )(parts, xws, degs, bg.reshape(1, -1), Wc, bc.reshape(1, -1))
    return out
